# Initial kernel scaffold; baseline (speedup 1.0000x reference)
import jax
import jax.numpy as jnp
from jax.experimental import pallas as pl


def kernel(x, gate_W, gate_b, expert_W, expert_b):
    raise NotImplementedError("write your pallas kernel here")


# speedup vs baseline: 2.3550x; 2.3550x over previous
import functools

import jax
import jax.numpy as jnp
from jax.experimental import pallas as pl
from jax.experimental.pallas import tpu as pltpu

INPUT_DIM = 1024
NUM_EXPERTS = 8
TOP_K = 2
TOKENS = 2048


def _moe_dense_kernel(x_ref, gw_ref, gb_ref, ew_ref, eb_ref, out_ref, c_ref):
    e = pl.program_id(0)

    @pl.when(e == 0)
    def _gating():
        x = x_ref[...]
        logits = jax.lax.dot_general(
            x, gw_ref[...], (((1,), (1,)), ((), ())),
            preferred_element_type=jnp.float32,
            precision=jax.lax.Precision.DEFAULT,
        ) + gb_ref[...]
        m = jnp.max(logits, axis=1, keepdims=True)
        ex = jnp.exp(logits - m)
        w = ex / jnp.sum(ex, axis=1, keepdims=True)
        col = jax.lax.broadcasted_iota(jnp.int32, w.shape, 1)
        rank = jnp.zeros(w.shape, jnp.int32)
        for ep in range(NUM_EXPERTS):
            wp = w[:, ep:ep + 1]
            rank = rank + (wp > w).astype(jnp.int32)
            rank = rank + ((wp == w) & (ep < col)).astype(jnp.int32)
        c_ref[...] = jnp.where(rank < TOP_K, w, 0.0)

    T = x_ref.shape[0]
    CHUNK = 256
    for tb in range(T // CHUNK):
        sl = pl.ds(tb * CHUNK, CHUNK)
        cc = c_ref[sl, :]
        ce = jnp.sum(
            jnp.where(
                jax.lax.broadcasted_iota(jnp.int32, cc.shape, 1) == e,
                cc, 0.0),
            axis=1, keepdims=True)
        y = jax.lax.dot_general(
            x_ref[sl, :], ew_ref[0], (((1,), (1,)), ((), ())),
            preferred_element_type=jnp.float32,
            precision=jax.lax.Precision.DEFAULT,
        ) + eb_ref[0]
        contrib = ce * y

        @pl.when(e == 0)
        def _init():
            out_ref[sl, :] = contrib

        @pl.when(e > 0)
        def _acc():
            out_ref[sl, :] += contrib


def kernel(x, gate_W, gate_b, expert_W, expert_b):
    T, D = x.shape
    E = gate_W.shape[0]
    return pl.pallas_call(
        _moe_dense_kernel,
        grid=(E,),
        in_specs=[
            pl.BlockSpec((T, D), lambda e: (0, 0)),
            pl.BlockSpec((E, D), lambda e: (0, 0)),
            pl.BlockSpec((1, E), lambda e: (0, 0)),
            pl.BlockSpec((1, D, D), lambda e: (e, 0, 0)),
            pl.BlockSpec((1, 1, D), lambda e: (e, 0, 0)),
        ],
        out_specs=pl.BlockSpec((T, D), lambda e: (0, 0)),
        out_shape=jax.ShapeDtypeStruct((T, D), jnp.float32),
        scratch_shapes=[pltpu.VMEM((T, E), jnp.float32)],
        compiler_params=pltpu.CompilerParams(
            dimension_semantics=("arbitrary",),
        ),
    )(x, gate_W, gate_b.reshape(1, E), expert_W, expert_b.reshape(E, 1, D))
